# Initial kernel scaffold; baseline (speedup 1.0000x reference)
import functools

import jax
import jax.numpy as jnp
from jax import lax
from jax.experimental import pallas as pl
from jax.experimental.pallas import tpu as pltpu
from jax.experimental.pallas import tpu_sc as plsc

NV = 100000
NC = 50000
E = 1600000
B = 16
H = 32

CH = 80
SUP = 2000
NJ = SUP // CH
BLK = 400


def _sc_conv(table, src2d, dst2d, ew2d, zeros_hbm, *, n_dst, d, feature_split,
             count_lane, split_off):
    ept = E // 16 if feature_split else E // 32
    nsup = ept // SUP
    rpt = n_dst // 16
    mesh = plsc.VectorSubcoreMesh(core_axis_name="c", subcore_axis_name="s")

    @functools.partial(
        pl.kernel,
        out_type=jax.ShapeDtypeStruct((2, n_dst, d), jnp.float32),
        mesh=mesh,
        scratch_types=[
            pltpu.VMEM_SHARED((n_dst, d), jnp.float32),
            pltpu.VMEM((NJ, CH), jnp.int32),
            pltpu.VMEM((NJ, CH), jnp.int32),
            pltpu.VMEM((NJ, CH), jnp.float32),
            pltpu.VMEM((CH, d), jnp.float32),
            pltpu.SemaphoreType.DMA,
        ],
    )
    def body(table_h, src_h, dst_h, ew_h, zeros_h, out_h,
             acc, src_v, dst_v, ew_v, rows_v, sem):
        c = lax.axis_index("c")
        s = lax.axis_index("s")
        lo = s * rpt
        pltpu.sync_copy(zeros_h, acc.at[pl.ds(lo, rpt)])
        plsc.subcore_barrier()

        if feature_split:
            row_base = s * (ept // CH)
        else:
            row_base = (c * 16 + s) * (ept // CH)
        lane = lax.iota(jnp.int32, 16)
        if count_lane is not None:
            cmask = lane == count_lane
        off_v = jnp.full((16,), c * split_off, jnp.int32)

        def sup_body(b, carry):
            row0 = row_base + b * NJ
            pltpu.sync_copy(src_h.at[pl.ds(row0, NJ)], src_v)
            pltpu.sync_copy(dst_h.at[pl.ds(row0, NJ)], dst_v)
            pltpu.sync_copy(ew_h.at[pl.ds(row0, NJ)], ew_v)
            if split_off:
                def adj(q, carry2):
                    j = q // (CH // 16)
                    k = q % (CH // 16)
                    src_v[j, pl.ds(k * 16, 16)] = (
                        src_v[j, pl.ds(k * 16, 16)] + off_v)
                    return carry2
                lax.fori_loop(0, NJ * (CH // 16), adj, 0)

            def chunk(j, carry2):
                pltpu.async_copy(table_h.at[src_v.at[j]], rows_v, sem).wait()

                def edge(i, carry3):
                    w = plsc.load_gather(
                        ew_v, [jnp.full((16,), j, jnp.int32),
                               jnp.full((16,), i, jnp.int32)])
                    if count_lane is not None:
                        w = jnp.where(cmask, 1.0, w)
                    for h in range(d // 16):
                        rows_v[i, pl.ds(h * 16, 16)] = (
                            rows_v[i, pl.ds(h * 16, 16)] * w)
                    return carry3
                lax.fori_loop(0, CH, edge, 0)
                pltpu.sync_copy(rows_v, acc.at[dst_v.at[j]], add=True)
                return carry2
            lax.fori_loop(0, NJ, chunk, 0)
            return carry
        lax.fori_loop(0, nsup, sup_body, 0)

        plsc.subcore_barrier()
        pltpu.sync_copy(acc.at[pl.ds(lo, rpt)], out_h.at[c, pl.ds(lo, rpt)])

    return body(table, src2d, dst2d, ew2d, zeros_hbm)


def _bn_stats(x):
    n, f = x.shape
    grid = n // BLK

    def body(x_ref, o_ref):
        @pl.when(pl.program_id(0) == 0)
        def _():
            o_ref[...] = jnp.zeros_like(o_ref)
        xb = x_ref[...]
        o_ref[...] += jnp.stack([jnp.sum(xb, 0), jnp.sum(xb * xb, 0)])

    return pl.pallas_call(
        body,
        grid=(grid,),
        in_specs=[pl.BlockSpec((BLK, f), lambda i: (i, 0))],
        out_specs=pl.BlockSpec((2, f), lambda i: (0, 0)),
        out_shape=jax.ShapeDtypeStruct((2, f), jnp.float32),
    )(x)


def _bn_pack(x, stats, g, b):
    n, f = x.shape
    grid = n // BLK

    def body(x_ref, s_ref, g_ref, b_ref, o_ref):
        m = s_ref[0:1, :] / n
        v = s_ref[1:2, :] / n - m * m
        xb = (x_ref[...] - m) * lax.rsqrt(v + 1e-5) * g_ref[...] + b_ref[...]
        o_ref[...] = jnp.concatenate(
            [xb, jnp.ones((BLK, 1), jnp.float32),
             jnp.zeros((BLK, 16 - f - 1), jnp.float32)], axis=1)

    return pl.pallas_call(
        body,
        grid=(grid,),
        in_specs=[
            pl.BlockSpec((BLK, f), lambda i: (i, 0)),
            pl.BlockSpec((2, f), lambda i: (0, 0)),
            pl.BlockSpec((1, f), lambda i: (0, 0)),
            pl.BlockSpec((1, f), lambda i: (0, 0)),
        ],
        out_specs=pl.BlockSpec((BLK, 16), lambda i: (i, 0)),
        out_shape=jax.ShapeDtypeStruct((n, 16), jnp.float32),
    )(x, stats, g.reshape(1, f), b.reshape(1, f))


def _l1_dense(p, table, wrel, brel, wroot, *, din, ddst):
    n = table.shape[0]
    grid = n // BLK

    def body(p_ref, t_ref, wrel_ref, brel_ref, wroot_ref,
             x2_ref, cinv_ref, st_ref):
        s = p_ref[0] + p_ref[1]
        cinv = 1.0 / jnp.maximum(s[:, din:din + 1], 1.0)
        mean = s[:, :din] * cinv
        z = (jnp.dot(mean, wrel_ref[...], preferred_element_type=jnp.float32)
             + brel_ref[...]
             + jnp.dot(t_ref[:, :ddst], wroot_ref[...],
                       preferred_element_type=jnp.float32))
        x2 = jnp.maximum(z, 0.0)
        x2_ref[...] = x2
        cinv_ref[...] = cinv
        @pl.when(pl.program_id(0) == 0)
        def _():
            st_ref[...] = jnp.zeros_like(st_ref)
        st_ref[...] += jnp.stack([jnp.sum(x2, 0), jnp.sum(x2 * x2, 0)])

    return pl.pallas_call(
        body,
        grid=(grid,),
        in_specs=[
            pl.BlockSpec((2, BLK, 16), lambda i: (0, i, 0)),
            pl.BlockSpec((BLK, 16), lambda i: (i, 0)),
            pl.BlockSpec(wrel.shape, lambda i: (0, 0)),
            pl.BlockSpec((1, H), lambda i: (0, 0)),
            pl.BlockSpec(wroot.shape, lambda i: (0, 0)),
        ],
        out_specs=[
            pl.BlockSpec((BLK, H), lambda i: (i, 0)),
            pl.BlockSpec((BLK, 1), lambda i: (i, 0)),
            pl.BlockSpec((2, H), lambda i: (0, 0)),
        ],
        out_shape=[
            jax.ShapeDtypeStruct((n, H), jnp.float32),
            jax.ShapeDtypeStruct((n, 1), jnp.float32),
            jax.ShapeDtypeStruct((2, H), jnp.float32),
        ],
    )(p, table, wrel, brel.reshape(1, H), wroot)


def _bn_apply2(x, stats, g, b, *, split):
    n = x.shape[0]
    grid = n // BLK

    def body(x_ref, s_ref, g_ref, b_ref, *o_refs):
        m = s_ref[0:1, :] / n
        v = s_ref[1:2, :] / n - m * m
        xb = (x_ref[...] - m) * lax.rsqrt(v + 1e-5) * g_ref[...] + b_ref[...]
        o_refs[0][...] = xb
        if split:
            o_refs[1][0] = xb[:, :16]
            o_refs[1][1] = xb[:, 16:]

    out_specs = [pl.BlockSpec((BLK, H), lambda i: (i, 0))]
    out_shape = [jax.ShapeDtypeStruct((n, H), jnp.float32)]
    if split:
        out_specs.append(pl.BlockSpec((2, BLK, 16), lambda i: (0, i, 0)))
        out_shape.append(jax.ShapeDtypeStruct((2, n, 16), jnp.float32))

    return pl.pallas_call(
        body,
        grid=(grid,),
        in_specs=[
            pl.BlockSpec((BLK, H), lambda i: (i, 0)),
            pl.BlockSpec((2, H), lambda i: (0, 0)),
            pl.BlockSpec((1, H), lambda i: (0, 0)),
            pl.BlockSpec((1, H), lambda i: (0, 0)),
        ],
        out_specs=out_specs,
        out_shape=out_shape,
    )(x, stats, g.reshape(1, H), b.reshape(1, H))


def _l2_pool(p, table, cinv, ids, wrel, brel, wroot, *, concat_halves):
    n = table.shape[0]
    grid = n // BLK

    def body(p_ref, t_ref, ci_ref, id_ref, wrel_ref, brel_ref, wroot_ref,
             o_ref, pacc, cacc):
        if concat_halves:
            s = jnp.concatenate([p_ref[0], p_ref[1]], axis=1)
        else:
            s = p_ref[0] + p_ref[1]
        mean = s * ci_ref[...]
        z = (jnp.dot(mean, wrel_ref[...], preferred_element_type=jnp.float32)
             + brel_ref[...]
             + jnp.dot(t_ref[...], wroot_ref[...],
                       preferred_element_type=jnp.float32))
        x2 = jnp.maximum(z, 0.0)
        oh = (id_ref[...] == lax.broadcasted_iota(jnp.int32, (1, B), 1)
              ).astype(jnp.float32)
        @pl.when(pl.program_id(0) == 0)
        def _():
            pacc[...] = jnp.zeros_like(pacc)
            cacc[...] = jnp.zeros_like(cacc)
        pacc[...] += lax.dot_general(oh, x2, (((0,), (0,)), ((), ())),
                                     preferred_element_type=jnp.float32)
        cacc[...] += lax.dot_general(oh, jnp.ones((BLK, 1), jnp.float32),
                                     (((0,), (0,)), ((), ())),
                                     preferred_element_type=jnp.float32)
        @pl.when(pl.program_id(0) == grid - 1)
        def _():
            o_ref[...] = pacc[...] / jnp.maximum(cacc[...], 1.0)

    return pl.pallas_call(
        body,
        grid=(grid,),
        in_specs=[
            pl.BlockSpec((2, BLK, p.shape[2]), lambda i: (0, i, 0)),
            pl.BlockSpec((BLK, H), lambda i: (i, 0)),
            pl.BlockSpec((BLK, 1), lambda i: (i, 0)),
            pl.BlockSpec((BLK, 1), lambda i: (i, 0)),
            pl.BlockSpec((H, H), lambda i: (0, 0)),
            pl.BlockSpec((1, H), lambda i: (0, 0)),
            pl.BlockSpec((H, H), lambda i: (0, 0)),
        ],
        out_specs=pl.BlockSpec((B, H), lambda i: (0, 0)),
        out_shape=jax.ShapeDtypeStruct((B, H), jnp.float32),
        scratch_shapes=[
            pltpu.VMEM((B, H), jnp.float32),
            pltpu.VMEM((B, 1), jnp.float32),
        ],
    )(p, table, cinv, ids, wrel, brel.reshape(1, H), wroot)


def kernel(var_feats, cstr_feats, edge_attr, edge_index, var_batch_el,
           cstr_batch_el, l1_bn_ng, l1_bn_nb, l1_bn_cg, l1_bn_cb, l1_n_Wrel,
           l1_n_brel, l1_n_Wroot, l1_c_Wrel, l1_c_brel, l1_c_Wroot, l2_bn_ng,
           l2_bn_nb, l2_bn_cg, l2_bn_cb, l2_n_Wrel, l2_n_brel, l2_n_Wroot,
           l2_c_Wrel, l2_c_brel, l2_c_Wroot):
    src2d = edge_index[0].reshape(E // CH, CH)
    dst2d = edge_index[1].reshape(E // CH, CH)
    ew2d = edge_attr.reshape(E // CH, CH)
    zeros_v16 = jnp.zeros((NV // 16, 16), jnp.float32)
    zeros_c16 = jnp.zeros((NC // 16, 16), jnp.float32)
    zeros_c32 = jnp.zeros((NC // 16, 32), jnp.float32)

    var_table = _bn_pack(var_feats, _bn_stats(var_feats), l1_bn_ng, l1_bn_nb)
    cstr_table = _bn_pack(cstr_feats, _bn_stats(cstr_feats),
                          l1_bn_cg, l1_bn_cb)

    p_v1 = _sc_conv(cstr_table, src2d, dst2d, ew2d, zeros_v16,
                    n_dst=NV, d=16, feature_split=False, count_lane=1,
                    split_off=0)
    p_c1 = _sc_conv(var_table, dst2d, src2d, ew2d, zeros_c16,
                    n_dst=NC, d=16, feature_split=False, count_lane=9,
                    split_off=0)

    xv2, cinv_v, vstats2 = _l1_dense(p_v1, var_table, l1_n_Wrel, l1_n_brel,
                                     l1_n_Wroot, din=1, ddst=9)
    xc2, cinv_c, cstats2 = _l1_dense(p_c1, cstr_table, l1_c_Wrel, l1_c_brel,
                                     l1_c_Wroot, din=9, ddst=1)

    (xv_bn2,) = _bn_apply2(xv2, vstats2, l2_bn_ng, l2_bn_nb, split=False)
    xc_bn2, xc_halves = _bn_apply2(xc2, cstats2, l2_bn_cg, l2_bn_cb,
                                   split=True)
    xc_stacked = xc_halves.reshape(2 * NC, 16)

    h_v2 = _sc_conv(xc_stacked, src2d, dst2d, ew2d, zeros_v16,
                    n_dst=NV, d=16, feature_split=True, count_lane=None,
                    split_off=NC)
    p_c2 = _sc_conv(xv_bn2, dst2d, src2d, ew2d, zeros_c32,
                    n_dst=NC, d=32, feature_split=False, count_lane=None,
                    split_off=0)

    xvp = _l2_pool(h_v2, xv_bn2, cinv_v, var_batch_el.reshape(NV, 1),
                   l2_n_Wrel, l2_n_brel, l2_n_Wroot, concat_halves=True)
    xcp = _l2_pool(p_c2, xc_bn2, cinv_c, cstr_batch_el.reshape(NC, 1),
                   l2_c_Wrel, l2_c_brel, l2_c_Wroot, concat_halves=False)
    return jnp.concatenate([xvp, xcp], axis=-1)


# speedup vs baseline: 10.1111x; 10.1111x over previous
import functools

import jax
import jax.numpy as jnp
from jax import lax
from jax.experimental import pallas as pl
from jax.experimental.pallas import tpu as pltpu
from jax.experimental.pallas import tpu_sc as plsc

NV = 100000
NC = 50000
E = 1600000
B = 16
H = 32

CH = 80
NJ = 25
SUP = NJ * CH
BLK = 400


def _ranges16(n):
    main = ((n // 16 + 7) // 8) * 8
    tail = n - 15 * main
    return main, tail


def _sc_conv(table0, table1, src2d, dst2d, ew2d, zeros_hbm, *, n_dst, d,
             feature_split, count_lane):
    ept = E // 16 if feature_split else E // 32
    nsup = ept // SUP
    rpt, rpt_tail = _ranges16(n_dst)
    mesh = plsc.VectorSubcoreMesh(core_axis_name="c", subcore_axis_name="s")

    @functools.partial(
        pl.kernel,
        out_type=jax.ShapeDtypeStruct((2, n_dst, d), jnp.float32),
        mesh=mesh,
        compiler_params=pltpu.CompilerParams(use_tc_tiling_on_sc=False),
        scratch_types=[
            pltpu.VMEM_SHARED((n_dst, d), jnp.float32),
            pltpu.VMEM((NJ, CH), jnp.int32),
            pltpu.VMEM((NJ, CH), jnp.int32),
            pltpu.VMEM((SUP,), jnp.float32),
            pltpu.VMEM((CH, d), jnp.float32),
            pltpu.SemaphoreType.DMA,
        ],
    )
    def body(table0_h, table1_h, src_h, dst_h, ew_h, zeros_h, out_h,
             acc, src_v, dst_v, ew_v, rows_v, sem):
        c = lax.axis_index("c")
        s = lax.axis_index("s")
        lo = s * rpt

        @pl.when(s < 15)
        def _():
            pltpu.sync_copy(zeros_h.at[pl.ds(0, rpt)], acc.at[pl.ds(lo, rpt)])

        @pl.when(s == 15)
        def _():
            pltpu.sync_copy(zeros_h.at[pl.ds(0, rpt_tail)],
                            acc.at[pl.ds(15 * rpt, rpt_tail)])
        plsc.subcore_barrier()

        if feature_split:
            sup_base = s * nsup
            e_base = s * ept
        else:
            sup_base = (c * 16 + s) * nsup
            e_base = (c * 16 + s) * ept
        lane = lax.iota(jnp.int32, 16)
        if count_lane is not None:
            cmask = lane == count_lane

        def run(table_h):
            def sup_body(b, carry):
                g = sup_base + b
                pltpu.sync_copy(src_h.at[g], src_v)
                pltpu.sync_copy(dst_h.at[g], dst_v)
                pltpu.sync_copy(ew_h.at[pl.ds(e_base + b * SUP, SUP)], ew_v)

                def chunk(j, carry2):
                    pltpu.async_copy(table_h.at[src_v.at[j]], rows_v,
                                     sem).wait()
                    jz = j * 0
                    for t in range(CH // 16):
                        w16 = ew_v[pl.ds(pl.multiple_of(j * CH + t * 16, 16),
                                         16)]
                        for u in range(16):
                            i = jz + t * 16 + u
                            w = jnp.full((16,), w16[u], jnp.float32)
                            if count_lane is not None:
                                w = jnp.where(cmask, 1.0, w)
                            for h in range(d // 16):
                                rows_v[i, pl.ds(h * 16, 16)] = (
                                    rows_v[i, pl.ds(h * 16, 16)] * w)
                    pltpu.sync_copy(rows_v, acc.at[dst_v.at[j]], add=True)
                    return carry2
                lax.fori_loop(0, NJ, chunk, 0)
                return carry
            lax.fori_loop(0, nsup, sup_body, 0)

        if feature_split:
            @pl.when(c == 0)
            def _():
                run(table0_h)

            @pl.when(c == 1)
            def _():
                run(table1_h)
        else:
            run(table0_h)

        plsc.subcore_barrier()

        @pl.when(s < 15)
        def _():
            pltpu.sync_copy(acc.at[pl.ds(lo, rpt)],
                            out_h.at[c, pl.ds(lo, rpt)])

        @pl.when(s == 15)
        def _():
            pltpu.sync_copy(acc.at[pl.ds(15 * rpt, rpt_tail)],
                            out_h.at[c, pl.ds(15 * rpt, rpt_tail)])

    return body(table0, table1, src2d, dst2d, ew2d, zeros_hbm)


def _bn_stats(x):
    n, f = x.shape
    grid = n // BLK

    def body(x_ref, o_ref):
        @pl.when(pl.program_id(0) == 0)
        def _():
            o_ref[...] = jnp.zeros_like(o_ref)
        xb = x_ref[...]
        o_ref[...] += jnp.stack([jnp.sum(xb, 0), jnp.sum(xb * xb, 0)])

    return pl.pallas_call(
        body,
        grid=(grid,),
        in_specs=[pl.BlockSpec((BLK, f), lambda i: (i, 0))],
        out_specs=pl.BlockSpec((2, f), lambda i: (0, 0)),
        out_shape=jax.ShapeDtypeStruct((2, f), jnp.float32),
    )(x)


def _bn_pack(x, stats, g, b):
    n, f = x.shape
    grid = n // BLK

    def body(x_ref, s_ref, g_ref, b_ref, o_ref):
        m = s_ref[0:1, :] / n
        v = s_ref[1:2, :] / n - m * m
        xb = (x_ref[...] - m) * lax.rsqrt(v + 1e-5) * g_ref[...] + b_ref[...]
        o_ref[...] = jnp.concatenate(
            [xb, jnp.ones((BLK, 1), jnp.float32),
             jnp.zeros((BLK, 16 - f - 1), jnp.float32)], axis=1)

    return pl.pallas_call(
        body,
        grid=(grid,),
        in_specs=[
            pl.BlockSpec((BLK, f), lambda i: (i, 0)),
            pl.BlockSpec((2, f), lambda i: (0, 0)),
            pl.BlockSpec((1, f), lambda i: (0, 0)),
            pl.BlockSpec((1, f), lambda i: (0, 0)),
        ],
        out_specs=pl.BlockSpec((BLK, 16), lambda i: (i, 0)),
        out_shape=jax.ShapeDtypeStruct((n, 16), jnp.float32),
    )(x, stats, g.reshape(1, f), b.reshape(1, f))


def _l1_dense(p, table, wrel, brel, wroot, *, din, ddst):
    n = table.shape[0]
    grid = n // BLK

    def body(p_ref, t_ref, wrel_ref, brel_ref, wroot_ref,
             x2_ref, cinv_ref, st_ref):
        s = p_ref[0] + p_ref[1]
        cinv = 1.0 / jnp.maximum(s[:, din:din + 1], 1.0)
        mean = s[:, :din] * cinv
        z = (jnp.dot(mean, wrel_ref[...], preferred_element_type=jnp.float32)
             + brel_ref[...]
             + jnp.dot(t_ref[:, :ddst], wroot_ref[...],
                       preferred_element_type=jnp.float32))
        x2 = jnp.maximum(z, 0.0)
        x2_ref[...] = x2
        cinv_ref[...] = cinv
        @pl.when(pl.program_id(0) == 0)
        def _():
            st_ref[...] = jnp.zeros_like(st_ref)
        st_ref[...] += jnp.stack([jnp.sum(x2, 0), jnp.sum(x2 * x2, 0)])

    return pl.pallas_call(
        body,
        grid=(grid,),
        in_specs=[
            pl.BlockSpec((2, BLK, 16), lambda i: (0, i, 0)),
            pl.BlockSpec((BLK, 16), lambda i: (i, 0)),
            pl.BlockSpec(wrel.shape, lambda i: (0, 0)),
            pl.BlockSpec((1, H), lambda i: (0, 0)),
            pl.BlockSpec(wroot.shape, lambda i: (0, 0)),
        ],
        out_specs=[
            pl.BlockSpec((BLK, H), lambda i: (i, 0)),
            pl.BlockSpec((BLK, 1), lambda i: (i, 0)),
            pl.BlockSpec((2, H), lambda i: (0, 0)),
        ],
        out_shape=[
            jax.ShapeDtypeStruct((n, H), jnp.float32),
            jax.ShapeDtypeStruct((n, 1), jnp.float32),
            jax.ShapeDtypeStruct((2, H), jnp.float32),
        ],
    )(p, table, wrel, brel.reshape(1, H), wroot)


def _bn_apply2(x, stats, g, b, *, split):
    n = x.shape[0]
    grid = n // BLK

    def body(x_ref, s_ref, g_ref, b_ref, *o_refs):
        m = s_ref[0:1, :] / n
        v = s_ref[1:2, :] / n - m * m
        xb = (x_ref[...] - m) * lax.rsqrt(v + 1e-5) * g_ref[...] + b_ref[...]
        o_refs[0][...] = xb
        if split:
            o_refs[1][0] = xb[:, :16]
            o_refs[1][1] = xb[:, 16:]

    out_specs = [pl.BlockSpec((BLK, H), lambda i: (i, 0))]
    out_shape = [jax.ShapeDtypeStruct((n, H), jnp.float32)]
    if split:
        out_specs.append(pl.BlockSpec((2, BLK, 16), lambda i: (0, i, 0)))
        out_shape.append(jax.ShapeDtypeStruct((2, n, 16), jnp.float32))

    return pl.pallas_call(
        body,
        grid=(grid,),
        in_specs=[
            pl.BlockSpec((BLK, H), lambda i: (i, 0)),
            pl.BlockSpec((2, H), lambda i: (0, 0)),
            pl.BlockSpec((1, H), lambda i: (0, 0)),
            pl.BlockSpec((1, H), lambda i: (0, 0)),
        ],
        out_specs=out_specs,
        out_shape=out_shape,
    )(x, stats, g.reshape(1, H), b.reshape(1, H))


def _l2_pool(p, table, cinv, ids, wrel, brel, wroot, *, concat_halves):
    n = table.shape[0]
    grid = n // BLK

    def body(p_ref, t_ref, ci_ref, id_ref, wrel_ref, brel_ref, wroot_ref,
             o_ref, pacc, cacc):
        if concat_halves:
            s = jnp.concatenate([p_ref[0], p_ref[1]], axis=1)
        else:
            s = p_ref[0] + p_ref[1]
        mean = s * ci_ref[...]
        z = (jnp.dot(mean, wrel_ref[...], preferred_element_type=jnp.float32)
             + brel_ref[...]
             + jnp.dot(t_ref[...], wroot_ref[...],
                       preferred_element_type=jnp.float32))
        x2 = jnp.maximum(z, 0.0)
        oh = (id_ref[...] == lax.broadcasted_iota(jnp.int32, (1, B), 1)
              ).astype(jnp.float32)
        @pl.when(pl.program_id(0) == 0)
        def _():
            pacc[...] = jnp.zeros_like(pacc)
            cacc[...] = jnp.zeros_like(cacc)
        pacc[...] += lax.dot_general(oh, x2, (((0,), (0,)), ((), ())),
                                     preferred_element_type=jnp.float32)
        cacc[...] += lax.dot_general(oh, jnp.ones((BLK, 1), jnp.float32),
                                     (((0,), (0,)), ((), ())),
                                     preferred_element_type=jnp.float32)
        @pl.when(pl.program_id(0) == grid - 1)
        def _():
            o_ref[...] = pacc[...] / jnp.maximum(cacc[...], 1.0)

    return pl.pallas_call(
        body,
        grid=(grid,),
        in_specs=[
            pl.BlockSpec((2, BLK, p.shape[2]), lambda i: (0, i, 0)),
            pl.BlockSpec((BLK, H), lambda i: (i, 0)),
            pl.BlockSpec((BLK, 1), lambda i: (i, 0)),
            pl.BlockSpec((BLK, 1), lambda i: (i, 0)),
            pl.BlockSpec((H, H), lambda i: (0, 0)),
            pl.BlockSpec((1, H), lambda i: (0, 0)),
            pl.BlockSpec((H, H), lambda i: (0, 0)),
        ],
        out_specs=pl.BlockSpec((B, H), lambda i: (0, 0)),
        out_shape=jax.ShapeDtypeStruct((B, H), jnp.float32),
        scratch_shapes=[
            pltpu.VMEM((B, H), jnp.float32),
            pltpu.VMEM((B, 1), jnp.float32),
        ],
    )(p, table, cinv, ids, wrel, brel.reshape(1, H), wroot)


def kernel(var_feats, cstr_feats, edge_attr, edge_index, var_batch_el,
           cstr_batch_el, l1_bn_ng, l1_bn_nb, l1_bn_cg, l1_bn_cb, l1_n_Wrel,
           l1_n_brel, l1_n_Wroot, l1_c_Wrel, l1_c_brel, l1_c_Wroot, l2_bn_ng,
           l2_bn_nb, l2_bn_cg, l2_bn_cb, l2_n_Wrel, l2_n_brel, l2_n_Wroot,
           l2_c_Wrel, l2_c_brel, l2_c_Wroot):
    src2d = edge_index[0].reshape(E // SUP, NJ, CH)
    dst2d = edge_index[1].reshape(E // SUP, NJ, CH)
    ew2d = edge_attr
    zeros_v16 = jnp.zeros((_ranges16(NV)[0], 16), jnp.float32)
    zeros_c16 = jnp.zeros((_ranges16(NC)[0], 16), jnp.float32)
    zeros_c32 = jnp.zeros((_ranges16(NC)[0], 32), jnp.float32)

    var_table = _bn_pack(var_feats, _bn_stats(var_feats), l1_bn_ng, l1_bn_nb)
    cstr_table = _bn_pack(cstr_feats, _bn_stats(cstr_feats),
                          l1_bn_cg, l1_bn_cb)

    p_v1 = _sc_conv(cstr_table, cstr_table, src2d, dst2d, ew2d, zeros_v16,
                    n_dst=NV, d=16, feature_split=False, count_lane=1)
    p_c1 = _sc_conv(var_table, var_table, dst2d, src2d, ew2d, zeros_c16,
                    n_dst=NC, d=16, feature_split=False, count_lane=9)

    xv2, cinv_v, vstats2 = _l1_dense(p_v1, var_table, l1_n_Wrel, l1_n_brel,
                                     l1_n_Wroot, din=1, ddst=9)
    xc2, cinv_c, cstats2 = _l1_dense(p_c1, cstr_table, l1_c_Wrel, l1_c_brel,
                                     l1_c_Wroot, din=9, ddst=1)

    (xv_bn2,) = _bn_apply2(xv2, vstats2, l2_bn_ng, l2_bn_nb, split=False)
    xc_bn2, xc_halves = _bn_apply2(xc2, cstats2, l2_bn_cg, l2_bn_cb,
                                   split=True)

    h_v2 = _sc_conv(xc_halves[0], xc_halves[1], src2d, dst2d, ew2d, zeros_v16,
                    n_dst=NV, d=16, feature_split=True, count_lane=None)
    p_c2 = _sc_conv(xv_bn2, xv_bn2, dst2d, src2d, ew2d, zeros_c32,
                    n_dst=NC, d=32, feature_split=False, count_lane=None)

    xvp = _l2_pool(h_v2, xv_bn2, cinv_v, var_batch_el.reshape(NV, 1),
                   l2_n_Wrel, l2_n_brel, l2_n_Wroot, concat_halves=True)
    xcp = _l2_pool(p_c2, xc_bn2, cinv_c, cstr_batch_el.reshape(NC, 1),
                   l2_c_Wrel, l2_c_brel, l2_c_Wroot, concat_halves=False)
    return jnp.concatenate([xvp, xcp], axis=-1)


# speedup vs baseline: 16.1567x; 1.5979x over previous
import functools

import jax
import jax.numpy as jnp
from jax import lax
from jax.experimental import pallas as pl
from jax.experimental.pallas import tpu as pltpu
from jax.experimental.pallas import tpu_sc as plsc

NV = 100000
NC = 50000
E = 1600000
B = 16
H = 32

CH = 80
NJ = 25
SUP = NJ * CH
BLK = 400


def _ranges16(n):
    main = ((n // 16 + 7) // 8) * 8
    tail = n - 15 * main
    return main, tail


def _sc_conv(table0, table1, src_flat, dst_flat, ew_flat, zeros_hbm, *,
             n_dst, d, feature_split, count_lane):
    ept = E // 16 if feature_split else E // 32
    nsup = ept // SUP
    rj = NJ if n_dst * d + 16 * (3 * SUP + NJ * CH * d) <= 2_000_000 else 5
    ngrp = NJ // rj
    rpt, rpt_tail = _ranges16(n_dst)
    mesh = plsc.VectorSubcoreMesh(core_axis_name="c", subcore_axis_name="s")

    @functools.partial(
        pl.kernel,
        out_type=jax.ShapeDtypeStruct((2, n_dst, d), jnp.float32),
        mesh=mesh,
        compiler_params=pltpu.CompilerParams(use_tc_tiling_on_sc=False),
        scratch_types=[
            pltpu.VMEM_SHARED((n_dst, d), jnp.float32),
            pltpu.VMEM((NJ, CH), jnp.int32),
            pltpu.VMEM((NJ, CH), jnp.int32),
            pltpu.VMEM((SUP,), jnp.float32),
            pltpu.VMEM((rj * CH, d), jnp.float32),
            pltpu.SemaphoreType.DMA,
            pltpu.SemaphoreType.DMA,
        ],
    )
    def body(table0_h, table1_h, src_h, dst_h, ew_h, zeros_h, out_h,
             acc, src_v, dst_v, ew_v, rows_v, gsem, ssem):
        c = lax.axis_index("c")
        s = lax.axis_index("s")
        lo = s * rpt

        @pl.when(s < 15)
        def _():
            pltpu.sync_copy(zeros_h.at[pl.ds(0, rpt)], acc.at[pl.ds(lo, rpt)])

        @pl.when(s == 15)
        def _():
            pltpu.sync_copy(zeros_h.at[pl.ds(0, rpt_tail)],
                            acc.at[pl.ds(15 * rpt, rpt_tail)])
        plsc.subcore_barrier()

        if feature_split:
            sup_base = s * nsup
            e_base = s * ept
        else:
            sup_base = (c * 16 + s) * nsup
            e_base = (c * 16 + s) * ept
        lane = lax.iota(jnp.int32, 16)
        if count_lane is not None:
            cmask = lane == count_lane

        def run(table_h):
            def sup_body(b, carry):
                g = sup_base + b
                pltpu.sync_copy(src_h.at[g], src_v)
                pltpu.sync_copy(dst_h.at[g], dst_v)
                pltpu.sync_copy(ew_h.at[pl.ds(e_base + b * SUP, SUP)], ew_v)

                def group(gg, carry2):
                    q0 = gg * rj
                    for k in range(rj):
                        pltpu.async_copy(
                            table_h.at[src_v.at[q0 + k]],
                            rows_v.at[pl.ds(k * CH, CH)], gsem)
                    pltpu.make_async_copy(
                        out_h.at[0, pl.ds(0, rj * CH)], rows_v, gsem).wait()

                    def grp(t, carry3):
                        w16 = ew_v[pl.ds(
                            pl.multiple_of(gg * rj * CH + t * 16, 16), 16)]
                        i0 = t * 16
                        for u in range(16):
                            w = jnp.full((16,), w16[u], jnp.float32)
                            if count_lane is not None:
                                w = jnp.where(cmask, 1.0, w)
                            for h in range(d // 16):
                                rows_v[i0 + u, pl.ds(h * 16, 16)] = (
                                    rows_v[i0 + u, pl.ds(h * 16, 16)] * w)
                        return carry3
                    lax.fori_loop(0, rj * CH // 16, grp, 0)

                    for k in range(rj):
                        pltpu.async_copy(
                            rows_v.at[pl.ds(k * CH, CH)],
                            acc.at[dst_v.at[q0 + k]], ssem, add=True)
                    pltpu.make_async_copy(
                        rows_v, acc.at[pl.ds(0, rj * CH)], ssem).wait()
                    return carry2
                lax.fori_loop(0, ngrp, group, 0)
                return carry
            lax.fori_loop(0, nsup, sup_body, 0)

        if feature_split:
            @pl.when(c == 0)
            def _():
                run(table0_h)

            @pl.when(c == 1)
            def _():
                run(table1_h)
        else:
            run(table0_h)

        plsc.subcore_barrier()

        @pl.when(s < 15)
        def _():
            pltpu.sync_copy(acc.at[pl.ds(lo, rpt)],
                            out_h.at[c, pl.ds(lo, rpt)])

        @pl.when(s == 15)
        def _():
            pltpu.sync_copy(acc.at[pl.ds(15 * rpt, rpt_tail)],
                            out_h.at[c, pl.ds(15 * rpt, rpt_tail)])

    return body(table0, table1, src_flat.reshape(E // SUP, NJ, CH),
                dst_flat.reshape(E // SUP, NJ, CH), ew_flat, zeros_hbm)


def _bn_stats(x):
    n, f = x.shape
    grid = n // BLK

    def body(x_ref, o_ref):
        @pl.when(pl.program_id(0) == 0)
        def _():
            o_ref[...] = jnp.zeros_like(o_ref)
        xb = x_ref[...]
        o_ref[...] += jnp.stack([jnp.sum(xb, 0), jnp.sum(xb * xb, 0)])

    return pl.pallas_call(
        body,
        grid=(grid,),
        in_specs=[pl.BlockSpec((BLK, f), lambda i: (i, 0))],
        out_specs=pl.BlockSpec((2, f), lambda i: (0, 0)),
        out_shape=jax.ShapeDtypeStruct((2, f), jnp.float32),
    )(x)


def _bn_pack(x, stats, g, b):
    n, f = x.shape
    grid = n // BLK

    def body(x_ref, s_ref, g_ref, b_ref, o_ref):
        m = s_ref[0:1, :] / n
        v = s_ref[1:2, :] / n - m * m
        xb = (x_ref[...] - m) * lax.rsqrt(v + 1e-5) * g_ref[...] + b_ref[...]
        o_ref[...] = jnp.concatenate(
            [xb, jnp.ones((BLK, 1), jnp.float32),
             jnp.zeros((BLK, 16 - f - 1), jnp.float32)], axis=1)

    return pl.pallas_call(
        body,
        grid=(grid,),
        in_specs=[
            pl.BlockSpec((BLK, f), lambda i: (i, 0)),
            pl.BlockSpec((2, f), lambda i: (0, 0)),
            pl.BlockSpec((1, f), lambda i: (0, 0)),
            pl.BlockSpec((1, f), lambda i: (0, 0)),
        ],
        out_specs=pl.BlockSpec((BLK, 16), lambda i: (i, 0)),
        out_shape=jax.ShapeDtypeStruct((n, 16), jnp.float32),
    )(x, stats, g.reshape(1, f), b.reshape(1, f))


def _l1_dense(p, table, wrel, brel, wroot, *, din, ddst):
    n = table.shape[0]
    grid = n // BLK

    def body(p_ref, t_ref, wrel_ref, brel_ref, wroot_ref,
             x2_ref, cinv_ref, st_ref):
        s = p_ref[0] + p_ref[1]
        cinv = 1.0 / jnp.maximum(s[:, din:din + 1], 1.0)
        mean = s[:, :din] * cinv
        z = (jnp.dot(mean, wrel_ref[...], preferred_element_type=jnp.float32)
             + brel_ref[...]
             + jnp.dot(t_ref[:, :ddst], wroot_ref[...],
                       preferred_element_type=jnp.float32))
        x2 = jnp.maximum(z, 0.0)
        x2_ref[...] = x2
        cinv_ref[...] = cinv
        @pl.when(pl.program_id(0) == 0)
        def _():
            st_ref[...] = jnp.zeros_like(st_ref)
        st_ref[...] += jnp.stack([jnp.sum(x2, 0), jnp.sum(x2 * x2, 0)])

    return pl.pallas_call(
        body,
        grid=(grid,),
        in_specs=[
            pl.BlockSpec((2, BLK, 16), lambda i: (0, i, 0)),
            pl.BlockSpec((BLK, 16), lambda i: (i, 0)),
            pl.BlockSpec(wrel.shape, lambda i: (0, 0)),
            pl.BlockSpec((1, H), lambda i: (0, 0)),
            pl.BlockSpec(wroot.shape, lambda i: (0, 0)),
        ],
        out_specs=[
            pl.BlockSpec((BLK, H), lambda i: (i, 0)),
            pl.BlockSpec((BLK, 1), lambda i: (i, 0)),
            pl.BlockSpec((2, H), lambda i: (0, 0)),
        ],
        out_shape=[
            jax.ShapeDtypeStruct((n, H), jnp.float32),
            jax.ShapeDtypeStruct((n, 1), jnp.float32),
            jax.ShapeDtypeStruct((2, H), jnp.float32),
        ],
    )(p, table, wrel, brel.reshape(1, H), wroot)


def _bn_apply2(x, stats, g, b, *, split):
    n = x.shape[0]
    grid = n // BLK

    def body(x_ref, s_ref, g_ref, b_ref, *o_refs):
        m = s_ref[0:1, :] / n
        v = s_ref[1:2, :] / n - m * m
        xb = (x_ref[...] - m) * lax.rsqrt(v + 1e-5) * g_ref[...] + b_ref[...]
        o_refs[0][...] = xb
        if split:
            o_refs[1][0] = xb[:, :16]
            o_refs[1][1] = xb[:, 16:]

    out_specs = [pl.BlockSpec((BLK, H), lambda i: (i, 0))]
    out_shape = [jax.ShapeDtypeStruct((n, H), jnp.float32)]
    if split:
        out_specs.append(pl.BlockSpec((2, BLK, 16), lambda i: (0, i, 0)))
        out_shape.append(jax.ShapeDtypeStruct((2, n, 16), jnp.float32))

    return pl.pallas_call(
        body,
        grid=(grid,),
        in_specs=[
            pl.BlockSpec((BLK, H), lambda i: (i, 0)),
            pl.BlockSpec((2, H), lambda i: (0, 0)),
            pl.BlockSpec((1, H), lambda i: (0, 0)),
            pl.BlockSpec((1, H), lambda i: (0, 0)),
        ],
        out_specs=out_specs,
        out_shape=out_shape,
    )(x, stats, g.reshape(1, H), b.reshape(1, H))


def _l2_pool(p, table, cinv, ids, wrel, brel, wroot, *, concat_halves):
    n = table.shape[0]
    grid = n // BLK

    def body(p_ref, t_ref, ci_ref, id_ref, wrel_ref, brel_ref, wroot_ref,
             o_ref, pacc, cacc):
        if concat_halves:
            s = jnp.concatenate([p_ref[0], p_ref[1]], axis=1)
        else:
            s = p_ref[0] + p_ref[1]
        mean = s * ci_ref[...]
        z = (jnp.dot(mean, wrel_ref[...], preferred_element_type=jnp.float32)
             + brel_ref[...]
             + jnp.dot(t_ref[...], wroot_ref[...],
                       preferred_element_type=jnp.float32))
        x2 = jnp.maximum(z, 0.0)
        oh = (id_ref[...] == lax.broadcasted_iota(jnp.int32, (1, B), 1)
              ).astype(jnp.float32)
        @pl.when(pl.program_id(0) == 0)
        def _():
            pacc[...] = jnp.zeros_like(pacc)
            cacc[...] = jnp.zeros_like(cacc)
        pacc[...] += lax.dot_general(oh, x2, (((0,), (0,)), ((), ())),
                                     preferred_element_type=jnp.float32)
        cacc[...] += lax.dot_general(oh, jnp.ones((BLK, 1), jnp.float32),
                                     (((0,), (0,)), ((), ())),
                                     preferred_element_type=jnp.float32)
        @pl.when(pl.program_id(0) == grid - 1)
        def _():
            o_ref[...] = pacc[...] / jnp.maximum(cacc[...], 1.0)

    return pl.pallas_call(
        body,
        grid=(grid,),
        in_specs=[
            pl.BlockSpec((2, BLK, p.shape[2]), lambda i: (0, i, 0)),
            pl.BlockSpec((BLK, H), lambda i: (i, 0)),
            pl.BlockSpec((BLK, 1), lambda i: (i, 0)),
            pl.BlockSpec((BLK, 1), lambda i: (i, 0)),
            pl.BlockSpec((H, H), lambda i: (0, 0)),
            pl.BlockSpec((1, H), lambda i: (0, 0)),
            pl.BlockSpec((H, H), lambda i: (0, 0)),
        ],
        out_specs=pl.BlockSpec((B, H), lambda i: (0, 0)),
        out_shape=jax.ShapeDtypeStruct((B, H), jnp.float32),
        scratch_shapes=[
            pltpu.VMEM((B, H), jnp.float32),
            pltpu.VMEM((B, 1), jnp.float32),
        ],
    )(p, table, cinv, ids, wrel, brel.reshape(1, H), wroot)


def kernel(var_feats, cstr_feats, edge_attr, edge_index, var_batch_el,
           cstr_batch_el, l1_bn_ng, l1_bn_nb, l1_bn_cg, l1_bn_cb, l1_n_Wrel,
           l1_n_brel, l1_n_Wroot, l1_c_Wrel, l1_c_brel, l1_c_Wroot, l2_bn_ng,
           l2_bn_nb, l2_bn_cg, l2_bn_cb, l2_n_Wrel, l2_n_brel, l2_n_Wroot,
           l2_c_Wrel, l2_c_brel, l2_c_Wroot):
    src2d = edge_index[0]
    dst2d = edge_index[1]
    ew2d = edge_attr
    zeros_v16 = jnp.zeros((_ranges16(NV)[0], 16), jnp.float32)
    zeros_c16 = jnp.zeros((_ranges16(NC)[0], 16), jnp.float32)
    zeros_c32 = jnp.zeros((_ranges16(NC)[0], 32), jnp.float32)

    var_table = _bn_pack(var_feats, _bn_stats(var_feats), l1_bn_ng, l1_bn_nb)
    cstr_table = _bn_pack(cstr_feats, _bn_stats(cstr_feats),
                          l1_bn_cg, l1_bn_cb)

    p_v1 = _sc_conv(cstr_table, cstr_table, src2d, dst2d, ew2d, zeros_v16,
                    n_dst=NV, d=16, feature_split=False, count_lane=1)
    p_c1 = _sc_conv(var_table, var_table, dst2d, src2d, ew2d, zeros_c16,
                    n_dst=NC, d=16, feature_split=False, count_lane=9)

    xv2, cinv_v, vstats2 = _l1_dense(p_v1, var_table, l1_n_Wrel, l1_n_brel,
                                     l1_n_Wroot, din=1, ddst=9)
    xc2, cinv_c, cstats2 = _l1_dense(p_c1, cstr_table, l1_c_Wrel, l1_c_brel,
                                     l1_c_Wroot, din=9, ddst=1)

    (xv_bn2,) = _bn_apply2(xv2, vstats2, l2_bn_ng, l2_bn_nb, split=False)
    xc_bn2, xc_halves = _bn_apply2(xc2, cstats2, l2_bn_cg, l2_bn_cb,
                                   split=True)

    h_v2 = _sc_conv(xc_halves[0], xc_halves[1], src2d, dst2d, ew2d, zeros_v16,
                    n_dst=NV, d=16, feature_split=True, count_lane=None)
    p_c2 = _sc_conv(xv_bn2, xv_bn2, dst2d, src2d, ew2d, zeros_c32,
                    n_dst=NC, d=32, feature_split=False, count_lane=None)

    xvp = _l2_pool(h_v2, xv_bn2, cinv_v, var_batch_el.reshape(NV, 1),
                   l2_n_Wrel, l2_n_brel, l2_n_Wroot, concat_halves=True)
    xcp = _l2_pool(p_c2, xc_bn2, cinv_c, cstr_batch_el.reshape(NC, 1),
                   l2_c_Wrel, l2_c_brel, l2_c_Wroot, concat_halves=False)
    return jnp.concatenate([xvp, xcp], axis=-1)


# speedup vs baseline: 22.5088x; 1.3932x over previous
import functools

import jax
import jax.numpy as jnp
from jax import lax
from jax.experimental import pallas as pl
from jax.experimental.pallas import tpu as pltpu
from jax.experimental.pallas import tpu_sc as plsc

NV = 100000
NC = 50000
E = 1600000
B = 16
H = 32

CH = 80
NJ = 25
SUP = NJ * CH
BLK = 2000


def _ranges16(n):
    main = ((n // 16 + 7) // 8) * 8
    tail = n - 15 * main
    return main, tail


def _sc_conv(table0, table1, src_flat, dst_flat, ew_flat, zeros_hbm, *,
             n_dst, d, feature_split, count_lane):
    ept = E // 16 if feature_split else E // 32
    nsup = ept // SUP
    rj = 5
    ngrp = NJ // rj
    L = rj * CH
    pipeline = n_dst * d + 16 * (3 * SUP + 2 * L * d) <= 2_060_000
    nbuf = 2 if pipeline else 1
    rpt, rpt_tail = _ranges16(n_dst)
    mesh = plsc.VectorSubcoreMesh(core_axis_name="c", subcore_axis_name="s")

    @functools.partial(
        pl.kernel,
        out_type=jax.ShapeDtypeStruct((2, n_dst, d), jnp.float32),
        mesh=mesh,
        compiler_params=pltpu.CompilerParams(use_tc_tiling_on_sc=False),
        scratch_types=[
            pltpu.VMEM_SHARED((n_dst, d), jnp.float32),
            pltpu.VMEM((NJ, CH), jnp.int32),
            pltpu.VMEM((NJ, CH), jnp.int32),
            pltpu.VMEM((SUP,), jnp.float32),
        ] + [pltpu.VMEM((L, d), jnp.float32)] * nbuf
          + [pltpu.SemaphoreType.DMA] * (2 * nbuf),
    )
    def body(table0_h, table1_h, src_h, dst_h, ew_h, zeros_h, out_h,
             acc, src_v, dst_v, ew_v, *rows_and_sems):
        if pipeline:
            rows_a, rows_b, gsa, gsb, ssa, ssb = rows_and_sems
        else:
            rows_a, gsa, ssa = rows_and_sems
        c = lax.axis_index("c")
        s = lax.axis_index("s")
        lo = s * rpt

        @pl.when(s < 15)
        def _():
            pltpu.sync_copy(zeros_h.at[pl.ds(0, rpt)], acc.at[pl.ds(lo, rpt)])

        @pl.when(s == 15)
        def _():
            pltpu.sync_copy(zeros_h.at[pl.ds(0, rpt_tail)],
                            acc.at[pl.ds(15 * rpt, rpt_tail)])
        plsc.subcore_barrier()

        if feature_split:
            sup_base = s * nsup
            e_base = s * ept
        else:
            sup_base = (c * 16 + s) * nsup
            e_base = (c * 16 + s) * ept
        lane = lax.iota(jnp.int32, 16)
        if count_lane is not None:
            cmask = lane == count_lane

        def run(table_h):
            def fire_g(q0, rows_r, gsem):
                for k in range(rj):
                    pltpu.async_copy(table_h.at[src_v.at[q0 + k]],
                                     rows_r.at[pl.ds(k * CH, CH)], gsem)

            def drain_g(rows_r, gsem):
                pltpu.make_async_copy(out_h.at[0, pl.ds(0, L)],
                                      rows_r, gsem).wait()

            def fire_s(q0, rows_r, ssem):
                for k in range(rj):
                    pltpu.async_copy(rows_r.at[pl.ds(k * CH, CH)],
                                     acc.at[dst_v.at[q0 + k]], ssem, add=True)

            def drain_s(rows_r, ssem):
                pltpu.make_async_copy(rows_r, acc.at[pl.ds(0, L)],
                                      ssem).wait()

            def process(gg, rows_r):
                @plsc.parallel_loop(0, L // 16, unroll=2)
                def _(t):
                    w16 = ew_v[pl.ds(
                        pl.multiple_of(gg * L + t * 16, 16), 16)]
                    i0 = t * 16
                    for u in range(16):
                        w = jnp.full((16,), w16[u], jnp.float32)
                        if count_lane is not None:
                            w = jnp.where(cmask, 1.0, w)
                        for h in range(d // 16):
                            rows_r[i0 + u, pl.ds(h * 16, 16)] = (
                                rows_r[i0 + u, pl.ds(h * 16, 16)] * w)

            def stage(b):
                g = sup_base + b
                pltpu.sync_copy(src_h.at[g], src_v)
                pltpu.sync_copy(dst_h.at[g], dst_v)
                pltpu.sync_copy(ew_h.at[pl.ds(e_base + b * SUP, SUP)], ew_v)

            if pipeline:
                def sup_body(b, carry):
                    stage(b)

                    def step(gg, rows_p, gsp, ssp, rows_q, gsq, ssq):
                        @pl.when(gg == 0)
                        def _():
                            fire_g(0, rows_p, gsp)
                        drain_g(rows_p, gsp)

                        @pl.when(jnp.logical_and(gg >= 1, gg + 1 < ngrp))
                        def _():
                            drain_s(rows_q, ssq)

                        @pl.when(gg + 1 < ngrp)
                        def _():
                            fire_g((gg + 1) * rj, rows_q, gsq)
                        process(gg, rows_p)
                        fire_s(gg * rj, rows_p, ssp)

                    def group(gg, carry2):
                        @pl.when(gg % 2 == 0)
                        def _():
                            step(gg, rows_a, gsa, ssa, rows_b, gsb, ssb)

                        @pl.when(gg % 2 == 1)
                        def _():
                            step(gg, rows_b, gsb, ssb, rows_a, gsa, ssa)
                        return carry2
                    lax.fori_loop(0, ngrp, group, 0)
                    drain_s(rows_b, ssb)
                    drain_s(rows_a, ssa)
                    return carry
                lax.fori_loop(0, nsup, sup_body, 0)
            else:
                def sup_body(b, carry):
                    stage(b)

                    def group(gg, carry2):
                        fire_g(gg * rj, rows_a, gsa)
                        drain_g(rows_a, gsa)
                        process(gg, rows_a)
                        fire_s(gg * rj, rows_a, ssa)
                        drain_s(rows_a, ssa)
                        return carry2
                    lax.fori_loop(0, ngrp, group, 0)
                    return carry
                lax.fori_loop(0, nsup, sup_body, 0)

        if feature_split:
            @pl.when(c == 0)
            def _():
                run(table0_h)

            @pl.when(c == 1)
            def _():
                run(table1_h)
        else:
            run(table0_h)

        plsc.subcore_barrier()

        @pl.when(s < 15)
        def _():
            pltpu.sync_copy(acc.at[pl.ds(lo, rpt)],
                            out_h.at[c, pl.ds(lo, rpt)])

        @pl.when(s == 15)
        def _():
            pltpu.sync_copy(acc.at[pl.ds(15 * rpt, rpt_tail)],
                            out_h.at[c, pl.ds(15 * rpt, rpt_tail)])

    return body(table0, table1, src_flat.reshape(E // SUP, NJ, CH),
                dst_flat.reshape(E // SUP, NJ, CH), ew_flat, zeros_hbm)


def _bn_stats(x):
    n, f = x.shape
    grid = n // BLK

    def body(x_ref, o_ref):
        @pl.when(pl.program_id(0) == 0)
        def _():
            o_ref[...] = jnp.zeros_like(o_ref)
        xb = x_ref[...]
        o_ref[...] += jnp.stack([jnp.sum(xb, 0), jnp.sum(xb * xb, 0)])

    return pl.pallas_call(
        body,
        grid=(grid,),
        in_specs=[pl.BlockSpec((BLK, f), lambda i: (i, 0))],
        out_specs=pl.BlockSpec((2, f), lambda i: (0, 0)),
        out_shape=jax.ShapeDtypeStruct((2, f), jnp.float32),
    )(x)


def _bn_pack(x, stats, g, b):
    n, f = x.shape
    grid = n // BLK

    def body(x_ref, s_ref, g_ref, b_ref, o_ref):
        m = s_ref[0:1, :] / n
        v = s_ref[1:2, :] / n - m * m
        xb = (x_ref[...] - m) * lax.rsqrt(v + 1e-5) * g_ref[...] + b_ref[...]
        o_ref[...] = jnp.concatenate(
            [xb, jnp.ones((BLK, 1), jnp.float32),
             jnp.zeros((BLK, 16 - f - 1), jnp.float32)], axis=1)

    return pl.pallas_call(
        body,
        grid=(grid,),
        in_specs=[
            pl.BlockSpec((BLK, f), lambda i: (i, 0)),
            pl.BlockSpec((2, f), lambda i: (0, 0)),
            pl.BlockSpec((1, f), lambda i: (0, 0)),
            pl.BlockSpec((1, f), lambda i: (0, 0)),
        ],
        out_specs=pl.BlockSpec((BLK, 16), lambda i: (i, 0)),
        out_shape=jax.ShapeDtypeStruct((n, 16), jnp.float32),
    )(x, stats, g.reshape(1, f), b.reshape(1, f))


def _l1_dense(p, table, wrel, brel, wroot, *, din, ddst):
    n = table.shape[0]
    grid = n // BLK

    def body(p_ref, t_ref, wrel_ref, brel_ref, wroot_ref,
             x2_ref, cinv_ref, st_ref):
        s = p_ref[0] + p_ref[1]
        cinv = 1.0 / jnp.maximum(s[:, din:din + 1], 1.0)
        mean = s[:, :din] * cinv
        z = (jnp.dot(mean, wrel_ref[...], preferred_element_type=jnp.float32)
             + brel_ref[...]
             + jnp.dot(t_ref[:, :ddst], wroot_ref[...],
                       preferred_element_type=jnp.float32))
        x2 = jnp.maximum(z, 0.0)
        x2_ref[...] = x2
        cinv_ref[...] = cinv
        @pl.when(pl.program_id(0) == 0)
        def _():
            st_ref[...] = jnp.zeros_like(st_ref)
        st_ref[...] += jnp.stack([jnp.sum(x2, 0), jnp.sum(x2 * x2, 0)])

    return pl.pallas_call(
        body,
        grid=(grid,),
        in_specs=[
            pl.BlockSpec((2, BLK, 16), lambda i: (0, i, 0)),
            pl.BlockSpec((BLK, 16), lambda i: (i, 0)),
            pl.BlockSpec(wrel.shape, lambda i: (0, 0)),
            pl.BlockSpec((1, H), lambda i: (0, 0)),
            pl.BlockSpec(wroot.shape, lambda i: (0, 0)),
        ],
        out_specs=[
            pl.BlockSpec((BLK, H), lambda i: (i, 0)),
            pl.BlockSpec((BLK, 1), lambda i: (i, 0)),
            pl.BlockSpec((2, H), lambda i: (0, 0)),
        ],
        out_shape=[
            jax.ShapeDtypeStruct((n, H), jnp.float32),
            jax.ShapeDtypeStruct((n, 1), jnp.float32),
            jax.ShapeDtypeStruct((2, H), jnp.float32),
        ],
    )(p, table, wrel, brel.reshape(1, H), wroot)


def _bn_apply2(x, stats, g, b, *, split):
    n = x.shape[0]
    grid = n // BLK

    def body(x_ref, s_ref, g_ref, b_ref, *o_refs):
        m = s_ref[0:1, :] / n
        v = s_ref[1:2, :] / n - m * m
        xb = (x_ref[...] - m) * lax.rsqrt(v + 1e-5) * g_ref[...] + b_ref[...]
        o_refs[0][...] = xb
        if split:
            o_refs[1][0] = xb[:, :16]
            o_refs[1][1] = xb[:, 16:]

    out_specs = [pl.BlockSpec((BLK, H), lambda i: (i, 0))]
    out_shape = [jax.ShapeDtypeStruct((n, H), jnp.float32)]
    if split:
        out_specs.append(pl.BlockSpec((2, BLK, 16), lambda i: (0, i, 0)))
        out_shape.append(jax.ShapeDtypeStruct((2, n, 16), jnp.float32))

    return pl.pallas_call(
        body,
        grid=(grid,),
        in_specs=[
            pl.BlockSpec((BLK, H), lambda i: (i, 0)),
            pl.BlockSpec((2, H), lambda i: (0, 0)),
            pl.BlockSpec((1, H), lambda i: (0, 0)),
            pl.BlockSpec((1, H), lambda i: (0, 0)),
        ],
        out_specs=out_specs,
        out_shape=out_shape,
    )(x, stats, g.reshape(1, H), b.reshape(1, H))


def _l2_pool(p, table, cinv, ids, wrel, brel, wroot, *, concat_halves):
    n = table.shape[0]
    grid = n // BLK

    def body(p_ref, t_ref, ci_ref, id_ref, wrel_ref, brel_ref, wroot_ref,
             o_ref, pacc, cacc):
        if concat_halves:
            s = jnp.concatenate([p_ref[0], p_ref[1]], axis=1)
        else:
            s = p_ref[0] + p_ref[1]
        mean = s * ci_ref[...]
        z = (jnp.dot(mean, wrel_ref[...], preferred_element_type=jnp.float32)
             + brel_ref[...]
             + jnp.dot(t_ref[...], wroot_ref[...],
                       preferred_element_type=jnp.float32))
        x2 = jnp.maximum(z, 0.0)
        oh = (id_ref[...] == lax.broadcasted_iota(jnp.int32, (1, B), 1)
              ).astype(jnp.float32)
        @pl.when(pl.program_id(0) == 0)
        def _():
            pacc[...] = jnp.zeros_like(pacc)
            cacc[...] = jnp.zeros_like(cacc)
        pacc[...] += lax.dot_general(oh, x2, (((0,), (0,)), ((), ())),
                                     preferred_element_type=jnp.float32)
        cacc[...] += lax.dot_general(oh, jnp.ones((BLK, 1), jnp.float32),
                                     (((0,), (0,)), ((), ())),
                                     preferred_element_type=jnp.float32)
        @pl.when(pl.program_id(0) == grid - 1)
        def _():
            o_ref[...] = pacc[...] / jnp.maximum(cacc[...], 1.0)

    return pl.pallas_call(
        body,
        grid=(grid,),
        in_specs=[
            pl.BlockSpec((2, BLK, p.shape[2]), lambda i: (0, i, 0)),
            pl.BlockSpec((BLK, H), lambda i: (i, 0)),
            pl.BlockSpec((BLK, 1), lambda i: (i, 0)),
            pl.BlockSpec((BLK, 1), lambda i: (i, 0)),
            pl.BlockSpec((H, H), lambda i: (0, 0)),
            pl.BlockSpec((1, H), lambda i: (0, 0)),
            pl.BlockSpec((H, H), lambda i: (0, 0)),
        ],
        out_specs=pl.BlockSpec((B, H), lambda i: (0, 0)),
        out_shape=jax.ShapeDtypeStruct((B, H), jnp.float32),
        scratch_shapes=[
            pltpu.VMEM((B, H), jnp.float32),
            pltpu.VMEM((B, 1), jnp.float32),
        ],
    )(p, table, cinv, ids, wrel, brel.reshape(1, H), wroot)


def kernel(var_feats, cstr_feats, edge_attr, edge_index, var_batch_el,
           cstr_batch_el, l1_bn_ng, l1_bn_nb, l1_bn_cg, l1_bn_cb, l1_n_Wrel,
           l1_n_brel, l1_n_Wroot, l1_c_Wrel, l1_c_brel, l1_c_Wroot, l2_bn_ng,
           l2_bn_nb, l2_bn_cg, l2_bn_cb, l2_n_Wrel, l2_n_brel, l2_n_Wroot,
           l2_c_Wrel, l2_c_brel, l2_c_Wroot):
    src2d = edge_index[0]
    dst2d = edge_index[1]
    ew2d = edge_attr
    zeros_v16 = jnp.zeros((_ranges16(NV)[0], 16), jnp.float32)
    zeros_c16 = jnp.zeros((_ranges16(NC)[0], 16), jnp.float32)
    zeros_c32 = jnp.zeros((_ranges16(NC)[0], 32), jnp.float32)

    var_table = _bn_pack(var_feats, _bn_stats(var_feats), l1_bn_ng, l1_bn_nb)
    cstr_table = _bn_pack(cstr_feats, _bn_stats(cstr_feats),
                          l1_bn_cg, l1_bn_cb)

    p_v1 = _sc_conv(cstr_table, cstr_table, src2d, dst2d, ew2d, zeros_v16,
                    n_dst=NV, d=16, feature_split=False, count_lane=1)
    p_c1 = _sc_conv(var_table, var_table, dst2d, src2d, ew2d, zeros_c16,
                    n_dst=NC, d=16, feature_split=False, count_lane=9)

    xv2, cinv_v, vstats2 = _l1_dense(p_v1, var_table, l1_n_Wrel, l1_n_brel,
                                     l1_n_Wroot, din=1, ddst=9)
    xc2, cinv_c, cstats2 = _l1_dense(p_c1, cstr_table, l1_c_Wrel, l1_c_brel,
                                     l1_c_Wroot, din=9, ddst=1)

    (xv_bn2,) = _bn_apply2(xv2, vstats2, l2_bn_ng, l2_bn_nb, split=False)
    xc_bn2, xc_halves = _bn_apply2(xc2, cstats2, l2_bn_cg, l2_bn_cb,
                                   split=True)

    h_v2 = _sc_conv(xc_halves[0], xc_halves[1], src2d, dst2d, ew2d, zeros_v16,
                    n_dst=NV, d=16, feature_split=True, count_lane=None)
    p_c2 = _sc_conv(xv_bn2, xv_bn2, dst2d, src2d, ew2d, zeros_c32,
                    n_dst=NC, d=32, feature_split=False, count_lane=None)

    xvp = _l2_pool(h_v2, xv_bn2, cinv_v, var_batch_el.reshape(NV, 1),
                   l2_n_Wrel, l2_n_brel, l2_n_Wroot, concat_halves=True)
    xcp = _l2_pool(p_c2, xc_bn2, cinv_c, cstr_batch_el.reshape(NC, 1),
                   l2_c_Wrel, l2_c_brel, l2_c_Wroot, concat_halves=False)
    return jnp.concatenate([xvp, xcp], axis=-1)


# speedup vs baseline: 24.9797x; 1.1098x over previous
import functools

import jax
import jax.numpy as jnp
from jax import lax
from jax.experimental import pallas as pl
from jax.experimental.pallas import tpu as pltpu
from jax.experimental.pallas import tpu_sc as plsc

NV = 100000
NC = 50000
E = 1600000
B = 16
H = 32

CH = 80
NJ = 25
SUP = NJ * CH
BLK = 2000


def _ranges16(n):
    main = ((n // 16 + 7) // 8) * 8
    tail = n - 15 * main
    return main, tail


def _sc_conv(table0, table1, src_flat, dst_flat, ew_flat, zeros_hbm, *,
             n_dst, d, feature_split, count_lane):
    ept = E // 16 if feature_split else E // 32
    nsup = ept // SUP
    rj = 5
    ngrp = NJ // rj
    L = rj * CH
    pipeline = n_dst * d + 16 * (6 * SUP + 2 * L * d) <= 2_060_000
    nbuf = 2 if pipeline else 1
    rpt, rpt_tail = _ranges16(n_dst)
    mesh = plsc.VectorSubcoreMesh(core_axis_name="c", subcore_axis_name="s")

    @functools.partial(
        pl.kernel,
        out_type=jax.ShapeDtypeStruct((2, n_dst, d), jnp.float32),
        mesh=mesh,
        compiler_params=pltpu.CompilerParams(use_tc_tiling_on_sc=False),
        scratch_types=[
            pltpu.VMEM_SHARED((n_dst, d), jnp.float32),
        ] + [pltpu.VMEM((NJ, CH), jnp.int32),
             pltpu.VMEM((NJ, CH), jnp.int32),
             pltpu.VMEM((SUP,), jnp.float32)] * nbuf
          + [pltpu.VMEM((L, d), jnp.float32)] * nbuf
          + [pltpu.SemaphoreType.DMA] * (3 * nbuf),
    )
    def body(table0_h, table1_h, src_h, dst_h, ew_h, zeros_h, out_h,
             acc, *bufs):
        if pipeline:
            (src_v, dst_v, ew_v, src1_v, dst1_v, ew1_v, rows_a, rows_b,
             gsa, gsb, ssa, ssb, st0, st1) = bufs
        else:
            src_v, dst_v, ew_v, rows_a, gsa, ssa, st0 = bufs
        c = lax.axis_index("c")
        s = lax.axis_index("s")
        lo = s * rpt

        @pl.when(s < 15)
        def _():
            pltpu.sync_copy(zeros_h.at[pl.ds(0, rpt)], acc.at[pl.ds(lo, rpt)])

        @pl.when(s == 15)
        def _():
            pltpu.sync_copy(zeros_h.at[pl.ds(0, rpt_tail)],
                            acc.at[pl.ds(15 * rpt, rpt_tail)])
        plsc.subcore_barrier()

        if feature_split:
            sup_base = s * nsup
            e_base = s * ept
        else:
            sup_base = (c * 16 + s) * nsup
            e_base = (c * 16 + s) * ept
        lane = lax.iota(jnp.int32, 16)
        if count_lane is not None:
            cmask = lane == count_lane

        def run(table_h):
            def fire_g(q0, src_r, rows_r, gsem):
                for k in range(rj):
                    pltpu.async_copy(table_h.at[src_r.at[q0 + k]],
                                     rows_r.at[pl.ds(k * CH, CH)], gsem)

            def drain_g(rows_r, gsem):
                pltpu.make_async_copy(out_h.at[0, pl.ds(0, L)],
                                      rows_r, gsem).wait()

            def fire_s(q0, dst_r, rows_r, ssem):
                for k in range(rj):
                    pltpu.async_copy(rows_r.at[pl.ds(k * CH, CH)],
                                     acc.at[dst_r.at[q0 + k]], ssem, add=True)

            def drain_s(rows_r, ssem):
                pltpu.make_async_copy(rows_r, acc.at[pl.ds(0, L)],
                                      ssem).wait()

            def process(gg, ew_r, rows_r):
                @plsc.parallel_loop(0, L // 16, unroll=2)
                def _(t):
                    w16 = ew_r[pl.ds(
                        pl.multiple_of(gg * L + t * 16, 16), 16)]
                    i0 = t * 16
                    for u in range(16):
                        w = jnp.full((16,), w16[u], jnp.float32)
                        if count_lane is not None:
                            w = jnp.where(cmask, 1.0, w)
                        for h in range(d // 16):
                            rows_r[i0 + u, pl.ds(h * 16, 16)] = (
                                rows_r[i0 + u, pl.ds(h * 16, 16)] * w)

            def fire_stage(b, src_r, dst_r, ew_r, stsem):
                g = sup_base + b
                pltpu.async_copy(src_h.at[g], src_r, stsem)
                pltpu.async_copy(dst_h.at[g], dst_r, stsem)
                pltpu.async_copy(ew_h.at[pl.ds(e_base + b * SUP, SUP)],
                                 ew_r, stsem)

            def drain_stage(src_r, dst_r, ew_r, stsem):
                pltpu.make_async_copy(src_h.at[sup_base], src_r,
                                      stsem).wait()
                pltpu.make_async_copy(dst_h.at[sup_base], dst_r,
                                      stsem).wait()
                pltpu.make_async_copy(ew_h.at[pl.ds(e_base, SUP)], ew_r,
                                      stsem).wait()

            if pipeline:
                def do_groups(src_r, dst_r, ew_r):
                    def step(gg, rows_p, gsp, ssp, rows_q, gsq, ssq):
                        @pl.when(gg == 0)
                        def _():
                            fire_g(0, src_r, rows_p, gsp)
                        drain_g(rows_p, gsp)

                        @pl.when(jnp.logical_and(gg >= 1, gg + 1 < ngrp))
                        def _():
                            drain_s(rows_q, ssq)

                        @pl.when(gg + 1 < ngrp)
                        def _():
                            fire_g((gg + 1) * rj, src_r, rows_q, gsq)
                        process(gg, ew_r, rows_p)
                        fire_s(gg * rj, dst_r, rows_p, ssp)

                    def group(gg, carry2):
                        @pl.when(gg % 2 == 0)
                        def _():
                            step(gg, rows_a, gsa, ssa, rows_b, gsb, ssb)

                        @pl.when(gg % 2 == 1)
                        def _():
                            step(gg, rows_b, gsb, ssb, rows_a, gsa, ssa)
                        return carry2
                    lax.fori_loop(0, ngrp, group, 0)
                    drain_s(rows_b, ssb)
                    drain_s(rows_a, ssa)

                fire_stage(0, src_v, dst_v, ew_v, st0)

                def sup_body(b, carry):
                    @pl.when(b % 2 == 0)
                    def _():
                        drain_stage(src_v, dst_v, ew_v, st0)

                        @pl.when(b + 1 < nsup)
                        def _():
                            fire_stage(b + 1, src1_v, dst1_v, ew1_v, st1)
                        do_groups(src_v, dst_v, ew_v)

                    @pl.when(b % 2 == 1)
                    def _():
                        drain_stage(src1_v, dst1_v, ew1_v, st1)

                        @pl.when(b + 1 < nsup)
                        def _():
                            fire_stage(b + 1, src_v, dst_v, ew_v, st0)
                        do_groups(src1_v, dst1_v, ew1_v)
                    return carry
                lax.fori_loop(0, nsup, sup_body, 0)
            else:
                def sup_body(b, carry):
                    g = sup_base + b
                    pltpu.sync_copy(src_h.at[g], src_v)
                    pltpu.sync_copy(dst_h.at[g], dst_v)
                    pltpu.sync_copy(ew_h.at[pl.ds(e_base + b * SUP, SUP)],
                                    ew_v)

                    def group(gg, carry2):
                        fire_g(gg * rj, src_v, rows_a, gsa)
                        drain_g(rows_a, gsa)
                        process(gg, ew_v, rows_a)
                        fire_s(gg * rj, dst_v, rows_a, ssa)
                        drain_s(rows_a, ssa)
                        return carry2
                    lax.fori_loop(0, ngrp, group, 0)
                    return carry
                lax.fori_loop(0, nsup, sup_body, 0)

        if feature_split:
            @pl.when(c == 0)
            def _():
                run(table0_h)

            @pl.when(c == 1)
            def _():
                run(table1_h)
        else:
            run(table0_h)

        plsc.subcore_barrier()

        @pl.when(s < 15)
        def _():
            pltpu.sync_copy(acc.at[pl.ds(lo, rpt)],
                            out_h.at[c, pl.ds(lo, rpt)])

        @pl.when(s == 15)
        def _():
            pltpu.sync_copy(acc.at[pl.ds(15 * rpt, rpt_tail)],
                            out_h.at[c, pl.ds(15 * rpt, rpt_tail)])

    return body(table0, table1, src_flat.reshape(E // SUP, NJ, CH),
                dst_flat.reshape(E // SUP, NJ, CH), ew_flat, zeros_hbm)


def _bn_stats(x):
    n, f = x.shape
    grid = n // BLK

    def body(x_ref, o_ref):
        @pl.when(pl.program_id(0) == 0)
        def _():
            o_ref[...] = jnp.zeros_like(o_ref)
        xb = x_ref[...]
        o_ref[...] += jnp.stack([jnp.sum(xb, 0), jnp.sum(xb * xb, 0)])

    return pl.pallas_call(
        body,
        grid=(grid,),
        in_specs=[pl.BlockSpec((BLK, f), lambda i: (i, 0))],
        out_specs=pl.BlockSpec((2, f), lambda i: (0, 0)),
        out_shape=jax.ShapeDtypeStruct((2, f), jnp.float32),
    )(x)


def _bn_pack(x, stats, g, b):
    n, f = x.shape
    grid = n // BLK

    def body(x_ref, s_ref, g_ref, b_ref, o_ref):
        m = s_ref[0:1, :] / n
        v = s_ref[1:2, :] / n - m * m
        xb = (x_ref[...] - m) * lax.rsqrt(v + 1e-5) * g_ref[...] + b_ref[...]
        o_ref[...] = jnp.concatenate(
            [xb, jnp.ones((BLK, 1), jnp.float32),
             jnp.zeros((BLK, 16 - f - 1), jnp.float32)], axis=1)

    return pl.pallas_call(
        body,
        grid=(grid,),
        in_specs=[
            pl.BlockSpec((BLK, f), lambda i: (i, 0)),
            pl.BlockSpec((2, f), lambda i: (0, 0)),
            pl.BlockSpec((1, f), lambda i: (0, 0)),
            pl.BlockSpec((1, f), lambda i: (0, 0)),
        ],
        out_specs=pl.BlockSpec((BLK, 16), lambda i: (i, 0)),
        out_shape=jax.ShapeDtypeStruct((n, 16), jnp.float32),
    )(x, stats, g.reshape(1, f), b.reshape(1, f))


def _l1_dense(p, table, wrel, brel, wroot, *, din, ddst):
    n = table.shape[0]
    grid = n // BLK

    def body(p_ref, t_ref, wrel_ref, brel_ref, wroot_ref,
             x2_ref, cinv_ref, st_ref):
        s = p_ref[0] + p_ref[1]
        cinv = 1.0 / jnp.maximum(s[:, din:din + 1], 1.0)
        mean = s[:, :din] * cinv
        z = (jnp.dot(mean, wrel_ref[...], preferred_element_type=jnp.float32)
             + brel_ref[...]
             + jnp.dot(t_ref[:, :ddst], wroot_ref[...],
                       preferred_element_type=jnp.float32))
        x2 = jnp.maximum(z, 0.0)
        x2_ref[...] = x2
        cinv_ref[...] = cinv
        @pl.when(pl.program_id(0) == 0)
        def _():
            st_ref[...] = jnp.zeros_like(st_ref)
        st_ref[...] += jnp.stack([jnp.sum(x2, 0), jnp.sum(x2 * x2, 0)])

    return pl.pallas_call(
        body,
        grid=(grid,),
        in_specs=[
            pl.BlockSpec((2, BLK, 16), lambda i: (0, i, 0)),
            pl.BlockSpec((BLK, 16), lambda i: (i, 0)),
            pl.BlockSpec(wrel.shape, lambda i: (0, 0)),
            pl.BlockSpec((1, H), lambda i: (0, 0)),
            pl.BlockSpec(wroot.shape, lambda i: (0, 0)),
        ],
        out_specs=[
            pl.BlockSpec((BLK, H), lambda i: (i, 0)),
            pl.BlockSpec((BLK, 1), lambda i: (i, 0)),
            pl.BlockSpec((2, H), lambda i: (0, 0)),
        ],
        out_shape=[
            jax.ShapeDtypeStruct((n, H), jnp.float32),
            jax.ShapeDtypeStruct((n, 1), jnp.float32),
            jax.ShapeDtypeStruct((2, H), jnp.float32),
        ],
    )(p, table, wrel, brel.reshape(1, H), wroot)


def _bn_apply2(x, stats, g, b, *, split):
    n = x.shape[0]
    grid = n // BLK

    def body(x_ref, s_ref, g_ref, b_ref, *o_refs):
        m = s_ref[0:1, :] / n
        v = s_ref[1:2, :] / n - m * m
        xb = (x_ref[...] - m) * lax.rsqrt(v + 1e-5) * g_ref[...] + b_ref[...]
        o_refs[0][...] = xb
        if split:
            o_refs[1][0] = xb[:, :16]
            o_refs[1][1] = xb[:, 16:]

    out_specs = [pl.BlockSpec((BLK, H), lambda i: (i, 0))]
    out_shape = [jax.ShapeDtypeStruct((n, H), jnp.float32)]
    if split:
        out_specs.append(pl.BlockSpec((2, BLK, 16), lambda i: (0, i, 0)))
        out_shape.append(jax.ShapeDtypeStruct((2, n, 16), jnp.float32))

    return pl.pallas_call(
        body,
        grid=(grid,),
        in_specs=[
            pl.BlockSpec((BLK, H), lambda i: (i, 0)),
            pl.BlockSpec((2, H), lambda i: (0, 0)),
            pl.BlockSpec((1, H), lambda i: (0, 0)),
            pl.BlockSpec((1, H), lambda i: (0, 0)),
        ],
        out_specs=out_specs,
        out_shape=out_shape,
    )(x, stats, g.reshape(1, H), b.reshape(1, H))


def _l2_pool(p, table, cinv, ids, wrel, brel, wroot, *, concat_halves):
    n = table.shape[0]
    grid = n // BLK

    def body(p_ref, t_ref, ci_ref, id_ref, wrel_ref, brel_ref, wroot_ref,
             o_ref, pacc, cacc):
        if concat_halves:
            s = jnp.concatenate([p_ref[0], p_ref[1]], axis=1)
        else:
            s = p_ref[0] + p_ref[1]
        mean = s * ci_ref[...]
        z = (jnp.dot(mean, wrel_ref[...], preferred_element_type=jnp.float32)
             + brel_ref[...]
             + jnp.dot(t_ref[...], wroot_ref[...],
                       preferred_element_type=jnp.float32))
        x2 = jnp.maximum(z, 0.0)
        oh = (id_ref[...] == lax.broadcasted_iota(jnp.int32, (1, B), 1)
              ).astype(jnp.float32)
        @pl.when(pl.program_id(0) == 0)
        def _():
            pacc[...] = jnp.zeros_like(pacc)
            cacc[...] = jnp.zeros_like(cacc)
        pacc[...] += lax.dot_general(oh, x2, (((0,), (0,)), ((), ())),
                                     preferred_element_type=jnp.float32)
        cacc[...] += lax.dot_general(oh, jnp.ones((BLK, 1), jnp.float32),
                                     (((0,), (0,)), ((), ())),
                                     preferred_element_type=jnp.float32)
        @pl.when(pl.program_id(0) == grid - 1)
        def _():
            o_ref[...] = pacc[...] / jnp.maximum(cacc[...], 1.0)

    return pl.pallas_call(
        body,
        grid=(grid,),
        in_specs=[
            pl.BlockSpec((2, BLK, p.shape[2]), lambda i: (0, i, 0)),
            pl.BlockSpec((BLK, H), lambda i: (i, 0)),
            pl.BlockSpec((BLK, 1), lambda i: (i, 0)),
            pl.BlockSpec((BLK, 1), lambda i: (i, 0)),
            pl.BlockSpec((H, H), lambda i: (0, 0)),
            pl.BlockSpec((1, H), lambda i: (0, 0)),
            pl.BlockSpec((H, H), lambda i: (0, 0)),
        ],
        out_specs=pl.BlockSpec((B, H), lambda i: (0, 0)),
        out_shape=jax.ShapeDtypeStruct((B, H), jnp.float32),
        scratch_shapes=[
            pltpu.VMEM((B, H), jnp.float32),
            pltpu.VMEM((B, 1), jnp.float32),
        ],
    )(p, table, cinv, ids, wrel, brel.reshape(1, H), wroot)


def kernel(var_feats, cstr_feats, edge_attr, edge_index, var_batch_el,
           cstr_batch_el, l1_bn_ng, l1_bn_nb, l1_bn_cg, l1_bn_cb, l1_n_Wrel,
           l1_n_brel, l1_n_Wroot, l1_c_Wrel, l1_c_brel, l1_c_Wroot, l2_bn_ng,
           l2_bn_nb, l2_bn_cg, l2_bn_cb, l2_n_Wrel, l2_n_brel, l2_n_Wroot,
           l2_c_Wrel, l2_c_brel, l2_c_Wroot):
    src2d = edge_index[0]
    dst2d = edge_index[1]
    ew2d = edge_attr
    zeros_v16 = jnp.zeros((_ranges16(NV)[0], 16), jnp.float32)
    zeros_c16 = jnp.zeros((_ranges16(NC)[0], 16), jnp.float32)
    zeros_c32 = jnp.zeros((_ranges16(NC)[0], 32), jnp.float32)

    var_table = _bn_pack(var_feats, _bn_stats(var_feats), l1_bn_ng, l1_bn_nb)
    cstr_table = _bn_pack(cstr_feats, _bn_stats(cstr_feats),
                          l1_bn_cg, l1_bn_cb)

    p_v1 = _sc_conv(cstr_table, cstr_table, src2d, dst2d, ew2d, zeros_v16,
                    n_dst=NV, d=16, feature_split=False, count_lane=1)
    p_c1 = _sc_conv(var_table, var_table, dst2d, src2d, ew2d, zeros_c16,
                    n_dst=NC, d=16, feature_split=False, count_lane=9)

    xv2, cinv_v, vstats2 = _l1_dense(p_v1, var_table, l1_n_Wrel, l1_n_brel,
                                     l1_n_Wroot, din=1, ddst=9)
    xc2, cinv_c, cstats2 = _l1_dense(p_c1, cstr_table, l1_c_Wrel, l1_c_brel,
                                     l1_c_Wroot, din=9, ddst=1)

    (xv_bn2,) = _bn_apply2(xv2, vstats2, l2_bn_ng, l2_bn_nb, split=False)
    xc_bn2, xc_halves = _bn_apply2(xc2, cstats2, l2_bn_cg, l2_bn_cb,
                                   split=True)

    h_v2 = _sc_conv(xc_halves[0], xc_halves[1], src2d, dst2d, ew2d, zeros_v16,
                    n_dst=NV, d=16, feature_split=True, count_lane=None)
    p_c2 = _sc_conv(xv_bn2, xv_bn2, dst2d, src2d, ew2d, zeros_c32,
                    n_dst=NC, d=32, feature_split=False, count_lane=None)

    xvp = _l2_pool(h_v2, xv_bn2, cinv_v, var_batch_el.reshape(NV, 1),
                   l2_n_Wrel, l2_n_brel, l2_n_Wroot, concat_halves=True)
    xcp = _l2_pool(p_c2, xc_bn2, cinv_c, cstr_batch_el.reshape(NC, 1),
                   l2_c_Wrel, l2_c_brel, l2_c_Wroot, concat_halves=False)
    return jnp.concatenate([xvp, xcp], axis=-1)
